# baseline (speedup 1.0000x reference)
import jax
import jax.numpy as jnp
from jax.experimental import pallas as pl
from jax.experimental.pallas import tpu as pltpu

_TIMESTEPS = 1000
_BETA_MIN, _BETA_MAX = 1e-4, 0.02

_P = 2
_BN = 512


def _body(scal_ref, x_ref, imt_ref, m_ref, d_ref, acc_ref):
    nb = pl.program_id(1)
    c1 = scal_ref[0]
    c2 = scal_ref[1]

    @pl.when(nb == 0)
    def _init():
        m_ref[...] = jnp.full_like(m_ref, -jnp.inf)
        d_ref[...] = jnp.zeros_like(d_ref)
        acc_ref[...] = jnp.zeros_like(acc_ref)

    imt = imt_ref[...]
    imt_b = imt.astype(jnp.bfloat16)
    dots = jax.lax.dot_general(
        x_ref[...].astype(jnp.bfloat16), imt_b,
        (((1,), (0,)), ((), ())),
        preferred_element_type=jnp.float32)
    i2 = jnp.sum(imt * imt, axis=0, keepdims=True)
    lt = jnp.transpose(c1 * dots - c2 * i2)

    m_old = m_ref[0]
    m_new = jnp.maximum(m_old, jnp.max(lt, axis=0, keepdims=True))
    alpha = jnp.exp(m_old - m_new)
    wt = jnp.exp(lt - m_new)
    d_ref[0] = d_ref[0] * alpha + jnp.sum(wt, axis=0, keepdims=True)
    acc_ref[0] = acc_ref[0] * alpha + jax.lax.dot_general(
        imt_b, wt.astype(jnp.bfloat16),
        (((1,), (0,)), ((), ())),
        preferred_element_type=jnp.float32)
    m_ref[0] = m_new


def kernel(x, images, t):
    B = x.shape[0]
    N = images.shape[0]
    D = x.shape[1] * x.shape[2] * x.shape[3]

    betas = jnp.linspace(_BETA_MIN, _BETA_MAX, _TIMESTEPS, dtype=jnp.float32)
    a_bar = jnp.cumprod(1.0 - betas)[t]
    at = jnp.sqrt(a_bar)
    bt2 = 1.0 - a_bar

    xf = x.reshape(B, D)
    imt = images.transpose(1, 2, 3, 0).reshape(D, N)

    nb_total = N // (_P * _BN)
    scal = jnp.stack([at / bt2, (at * at) / (2.0 * bt2)])

    m_p, d_p, acc_p = pl.pallas_call(
        _body,
        grid=(_P, nb_total),
        in_specs=[
            pl.BlockSpec(memory_space=pltpu.SMEM),
            pl.BlockSpec((B, D), lambda p, nb: (0, 0)),
            pl.BlockSpec((D, _BN), lambda p, nb: (0, p * nb_total + nb)),
        ],
        out_specs=[
            pl.BlockSpec((1, 1, B), lambda p, nb: (p, 0, 0)),
            pl.BlockSpec((1, 1, B), lambda p, nb: (p, 0, 0)),
            pl.BlockSpec((1, D, B), lambda p, nb: (p, 0, 0)),
        ],
        out_shape=[
            jax.ShapeDtypeStruct((_P, 1, B), jnp.float32),
            jax.ShapeDtypeStruct((_P, 1, B), jnp.float32),
            jax.ShapeDtypeStruct((_P, D, B), jnp.float32),
        ],
        compiler_params=pltpu.CompilerParams(
            dimension_semantics=("parallel", "arbitrary"),
        ),
        name="ideal_score_online_softmax",
    )(scal, xf, imt)

    m_p = m_p[:, 0, :]
    d_p = d_p[:, 0, :]
    m = jnp.max(m_p, axis=0)
    s = jnp.exp(m_p - m[None, :])
    denom = jnp.sum(d_p * s, axis=0)
    w_img = jnp.sum(acc_p * s[:, None, :], axis=0)
    score = -((xf - at * (w_img.T / denom[:, None])) / bt2)
    return score.reshape(x.shape)


# speedup vs baseline: 1.1833x; 1.1833x over previous
import jax
import jax.numpy as jnp
from jax.experimental import pallas as pl
from jax.experimental.pallas import tpu as pltpu

_TIMESTEPS = 1000
_BETA_MIN, _BETA_MAX = 1e-4, 0.02

_P = 2
_BN = 1024


def _body(scal_ref, x_ref, imt_ref, m_ref, d_ref, acc_ref):
    nb = pl.program_id(1)
    c1 = scal_ref[0]
    c2 = scal_ref[1]

    @pl.when(nb == 0)
    def _init():
        m_ref[...] = jnp.full_like(m_ref, -jnp.inf)
        d_ref[...] = jnp.zeros_like(d_ref)
        acc_ref[...] = jnp.zeros_like(acc_ref)

    imt = imt_ref[...]
    imt_b = imt.astype(jnp.bfloat16)
    dots = jax.lax.dot_general(
        x_ref[...].astype(jnp.bfloat16), imt_b,
        (((1,), (0,)), ((), ())),
        preferred_element_type=jnp.float32)
    i2 = jnp.sum(imt * imt, axis=0, keepdims=True)
    logits = c1 * dots - c2 * i2

    m_old = m_ref[0]
    m_new = jnp.maximum(m_old, jnp.max(logits, axis=1, keepdims=True))
    alpha = jnp.exp(m_old - m_new)
    w = jnp.exp(logits - m_new)
    d_ref[0] = d_ref[0] * alpha + jnp.sum(w, axis=1, keepdims=True)
    acc_ref[0] = acc_ref[0] * alpha + jax.lax.dot_general(
        w.astype(jnp.bfloat16), imt_b,
        (((1,), (1,)), ((), ())),
        preferred_element_type=jnp.float32)
    m_ref[0] = m_new


def kernel(x, images, t):
    B = x.shape[0]
    N = images.shape[0]
    D = x.shape[1] * x.shape[2] * x.shape[3]

    betas = jnp.linspace(_BETA_MIN, _BETA_MAX, _TIMESTEPS, dtype=jnp.float32)
    a_bar = jnp.cumprod(1.0 - betas)[t]
    at = jnp.sqrt(a_bar)
    bt2 = 1.0 - a_bar

    xf = x.reshape(B, D)
    imt = images.transpose(1, 2, 3, 0).reshape(D, N)

    nb_total = N // (_P * _BN)
    scal = jnp.stack([at / bt2, (at * at) / (2.0 * bt2)])

    m_p, d_p, acc_p = pl.pallas_call(
        _body,
        grid=(_P, nb_total),
        in_specs=[
            pl.BlockSpec(memory_space=pltpu.SMEM),
            pl.BlockSpec((B, D), lambda p, nb: (0, 0)),
            pl.BlockSpec((D, _BN), lambda p, nb: (0, p * nb_total + nb)),
        ],
        out_specs=[
            pl.BlockSpec((1, B, 1), lambda p, nb: (p, 0, 0)),
            pl.BlockSpec((1, B, 1), lambda p, nb: (p, 0, 0)),
            pl.BlockSpec((1, B, D), lambda p, nb: (p, 0, 0)),
        ],
        out_shape=[
            jax.ShapeDtypeStruct((_P, B, 1), jnp.float32),
            jax.ShapeDtypeStruct((_P, B, 1), jnp.float32),
            jax.ShapeDtypeStruct((_P, B, D), jnp.float32),
        ],
        compiler_params=pltpu.CompilerParams(
            dimension_semantics=("parallel", "arbitrary"),
        ),
        name="ideal_score_online_softmax",
    )(scal, xf, imt)

    m_p = m_p[:, :, 0]
    d_p = d_p[:, :, 0]
    m = jnp.max(m_p, axis=0)
    s = jnp.exp(m_p - m[None, :])
    denom = jnp.sum(d_p * s, axis=0)
    w_img = jnp.sum(acc_p * s[:, :, None], axis=0)
    score = -((xf - at * (w_img / denom[:, None])) / bt2)
    return score.reshape(x.shape)


# speedup vs baseline: 1.2323x; 1.0414x over previous
import jax
import jax.numpy as jnp
from jax.experimental import pallas as pl
from jax.experimental.pallas import tpu as pltpu

_TIMESTEPS = 1000
_BETA_MIN, _BETA_MAX = 1e-4, 0.02

_P = 2
_BN = 2048
_BC = 1024


def _body(scal_ref, x_ref, imt_ref, m_ref, d_ref, acc_ref):
    nb = pl.program_id(1)
    c1 = scal_ref[0]
    c2 = scal_ref[1]

    @pl.when(nb == 0)
    def _init():
        m_ref[...] = jnp.full_like(m_ref, -jnp.inf)
        d_ref[...] = jnp.zeros_like(d_ref)
        acc_ref[...] = jnp.zeros_like(acc_ref)

    xb = x_ref[...].astype(jnp.bfloat16)
    for s in range(_BN // _BC):
        imt = imt_ref[:, s * _BC:(s + 1) * _BC]
        imt_b = imt.astype(jnp.bfloat16)
        dots = jax.lax.dot_general(
            xb, imt_b, (((1,), (0,)), ((), ())),
            preferred_element_type=jnp.float32)
        i2 = jnp.sum(imt * imt, axis=0, keepdims=True)
        logits = c1 * dots - c2 * i2

        m_old = m_ref[0]
        m_new = jnp.maximum(m_old, jnp.max(logits, axis=1, keepdims=True))
        alpha = jnp.exp(m_old - m_new)
        w = jnp.exp(logits - m_new)
        d_ref[0] = d_ref[0] * alpha + jnp.sum(w, axis=1, keepdims=True)
        acc_ref[0] = acc_ref[0] * alpha + jax.lax.dot_general(
            w.astype(jnp.bfloat16), imt_b,
            (((1,), (1,)), ((), ())),
            preferred_element_type=jnp.float32)
        m_ref[0] = m_new


def kernel(x, images, t):
    B = x.shape[0]
    N = images.shape[0]
    D = x.shape[1] * x.shape[2] * x.shape[3]

    betas = jnp.linspace(_BETA_MIN, _BETA_MAX, _TIMESTEPS, dtype=jnp.float32)
    a_bar = jnp.cumprod(1.0 - betas)[t]
    at = jnp.sqrt(a_bar)
    bt2 = 1.0 - a_bar

    xf = x.reshape(B, D)
    imt = images.transpose(1, 2, 3, 0).reshape(D, N)

    nb_total = N // (_P * _BN)
    scal = jnp.stack([at / bt2, (at * at) / (2.0 * bt2)])

    m_p, d_p, acc_p = pl.pallas_call(
        _body,
        grid=(_P, nb_total),
        in_specs=[
            pl.BlockSpec(memory_space=pltpu.SMEM),
            pl.BlockSpec((B, D), lambda p, nb: (0, 0)),
            pl.BlockSpec((D, _BN), lambda p, nb: (0, p * nb_total + nb)),
        ],
        out_specs=[
            pl.BlockSpec((1, B, 1), lambda p, nb: (p, 0, 0)),
            pl.BlockSpec((1, B, 1), lambda p, nb: (p, 0, 0)),
            pl.BlockSpec((1, B, D), lambda p, nb: (p, 0, 0)),
        ],
        out_shape=[
            jax.ShapeDtypeStruct((_P, B, 1), jnp.float32),
            jax.ShapeDtypeStruct((_P, B, 1), jnp.float32),
            jax.ShapeDtypeStruct((_P, B, D), jnp.float32),
        ],
        compiler_params=pltpu.CompilerParams(
            dimension_semantics=("parallel", "arbitrary"),
            vmem_limit_bytes=56 * 1024 * 1024,
        ),
        name="ideal_score_online_softmax",
    )(scal, xf, imt)

    m_p = m_p[:, :, 0]
    d_p = d_p[:, :, 0]
    m = jnp.max(m_p, axis=0)
    s = jnp.exp(m_p - m[None, :])
    denom = jnp.sum(d_p * s, axis=0)
    w_img = jnp.sum(acc_p * s[:, :, None], axis=0)
    score = -((xf - at * (w_img / denom[:, None])) / bt2)
    return score.reshape(x.shape)
